# baseline (speedup 1.0000x reference)
import numpy as np
import jax
import jax.numpy as jnp
from jax import lax
from jax.experimental import pallas as pl
from jax.experimental.pallas import tpu as pltpu

_TEMP = 10.0
_PERCENT_ON = 0.1
_B, _C, _H, _W = 64, 96, 56, 56
_N = _C * _H * _W
_K = int(round(_N * _PERCENT_ON))
_LANES = 128
_SUB = _N // _LANES
_ROWS = 4
_NCH = 21
_CH = _SUB // _NCH

_gumbel_cache = None


def _gumbel():
    global _gumbel_cache
    if _gumbel_cache is None:
        gkey = jax.random.fold_in(jax.random.key(0), 1)
        u = jax.random.uniform(gkey, (_B, _N), minval=1e-9, maxval=1.0)
        _gumbel_cache = -jnp.log(-jnp.log(u))
    return _gumbel_cache


def _allsum(acc):
    t = acc[0:8].astype(jnp.int32) + acc[8:16].astype(jnp.int32)
    for sh in (64, 32, 16, 8, 4, 2, 1):
        t = t + pltpu.roll(t, sh, 1)
    for sh in (4, 2, 1):
        t = t + pltpu.roll(t, sh, 0)
    return t


def _count4(ref, cands16, strict=False):
    accs = [jnp.zeros((16, _LANES), jnp.int16) for _ in range(_ROWS)]
    nsub = _CH // 16
    for c in range(_NCH):
        for r in range(_ROWS):
            sl = ref[r, pl.ds(c * _CH, _CH), :].reshape(nsub, 16, _LANES)
            m = (sl > cands16[r]) if strict else (sl >= cands16[r])
            mi = m.astype(jnp.int16)
            parts = [mi[j] for j in range(nsub)]
            while len(parts) > 1:
                parts = [parts[i] + parts[i + 1] if i + 1 < len(parts)
                         else parts[i] for i in range(0, len(parts), 2)]
            accs[r] = accs[r] + parts[0]
    return [_allsum(a) for a in accs]


def _bisect4(ref, targets):
    zero16 = [jnp.zeros((1, _LANES), jnp.int16)] * _ROWS
    cnt0 = _count4(ref, zero16)
    res0 = tuple(
        jnp.where(cnt0[r] >= targets[r],
                  jnp.zeros((8, _LANES), jnp.int32),
                  jnp.full((8, _LANES), -32768, jnp.int32))
        for r in range(_ROWS))
    bit0 = jnp.full((8, _LANES), 2**14, jnp.int32)

    def body(_, carry):
        res, bitval = carry
        cands = [res[r] | bitval for r in range(_ROWS)]
        cands16 = [c[0:1].astype(jnp.int16) for c in cands]
        cnts = _count4(ref, cands16)
        res = tuple(
            jnp.where(cnts[r] >= targets[r], cands[r], res[r])
            for r in range(_ROWS))
        return res, lax.shift_right_logical(bitval, 1)

    res, _ = lax.fori_loop(0, 15, body, (res0, bit0))
    return res


def _kwinners_body(x_ref, g_ref, o_ref, hi_ref, lo_ref):
    x = x_ref[...]
    pert = x / _TEMP + g_ref[...]
    s = lax.bitcast_convert_type(pert, jnp.int32)
    v = jnp.where(s < 0, s ^ jnp.int32(0x7FFFFFFF), s)
    hi_ref[...] = (v >> 16).astype(jnp.int16)
    lo_ref[...] = ((v & jnp.int32(0xFFFF)) ^ jnp.int32(0x8000)).astype(jnp.int16)

    kvec = jnp.full((8, _LANES), _K, jnp.int32)
    hi_thr = _bisect4(hi_ref, [kvec] * _ROWS)
    hi16 = [hi_thr[r][0:1].astype(jnp.int16) for r in range(_ROWS)]
    cgt = _count4(hi_ref, hi16, strict=True)
    k2 = [kvec - cgt[r] for r in range(_ROWS)]

    for r in range(_ROWS):
        lo_ref[r] = jnp.where(hi_ref[r] == hi16[r], lo_ref[r],
                              jnp.int16(-32768))

    lo_thr = _bisect4(lo_ref, k2)

    for r in range(_ROWS):
        h16 = hi16[r]
        l16 = lo_thr[r][0:1].astype(jnp.int16)
        keep = (hi_ref[r] > h16) | ((hi_ref[r] == h16) & (lo_ref[r] >= l16))
        o_ref[r] = jnp.where(keep, x_ref[r], jnp.float32(0.0))


def kernel(x):
    g = _gumbel()
    x3 = x.reshape(_B, _SUB, _LANES)
    g3 = g.reshape(_B, _SUB, _LANES)
    out = pl.pallas_call(
        _kwinners_body,
        grid=(_B // _ROWS,),
        in_specs=[
            pl.BlockSpec((_ROWS, _SUB, _LANES), lambda i: (i, 0, 0)),
            pl.BlockSpec((_ROWS, _SUB, _LANES), lambda i: (i, 0, 0)),
        ],
        out_specs=pl.BlockSpec((_ROWS, _SUB, _LANES), lambda i: (i, 0, 0)),
        out_shape=jax.ShapeDtypeStruct((_B, _SUB, _LANES), jnp.float32),
        scratch_shapes=[
            pltpu.VMEM((_ROWS, _SUB, _LANES), jnp.int16),
            pltpu.VMEM((_ROWS, _SUB, _LANES), jnp.int16),
        ],
    )(x3, g3)
    return out.reshape(_B, _C, _H, _W)


# speedup vs baseline: 1.1547x; 1.1547x over previous
import numpy as np
import jax
import jax.numpy as jnp
from jax import lax
from jax.experimental import pallas as pl
from jax.experimental.pallas import tpu as pltpu

_TEMP = 10.0
_PERCENT_ON = 0.1
_B, _C, _H, _W = 64, 96, 56, 56
_N = _C * _H * _W
_K = int(round(_N * _PERCENT_ON))
_LANES = 128
_SUB = _N // _LANES
_ROWS = 4
_NCH = 21
_CH = _SUB // _NCH

_gumbel_cache = None


def _gumbel():
    global _gumbel_cache
    if _gumbel_cache is None:
        gkey = jax.random.fold_in(jax.random.key(0), 1)
        u = jax.random.uniform(gkey, (_B, _N), minval=1e-9, maxval=1.0)
        _gumbel_cache = -jnp.log(-jnp.log(u))
    return _gumbel_cache


def _count4(ref, cands, strict=False):
    cands16 = [c.astype(jnp.int16) for c in cands]
    accs = [jnp.zeros((16, _LANES), jnp.int16) for _ in range(_ROWS)]
    nsub = _CH // 16
    for c in range(_NCH):
        for r in range(_ROWS):
            sl = ref[r, pl.ds(c * _CH, _CH), :].reshape(nsub, 16, _LANES)
            m = (sl > cands16[r]) if strict else (sl >= cands16[r])
            mi = m.astype(jnp.int16)
            parts = [mi[j] for j in range(nsub)]
            while len(parts) > 1:
                parts = [parts[i] + parts[i + 1] if i + 1 < len(parts)
                         else parts[i] for i in range(0, len(parts), 2)]
            accs[r] = accs[r] + parts[0]
    return [jnp.sum(a.astype(jnp.int32)) for a in accs]


def _bisect4(ref, targets):
    cnt0 = _count4(ref, [jnp.int32(0)] * _ROWS)
    res0 = tuple(
        jnp.where(cnt0[r] >= targets[r], jnp.int32(0), jnp.int32(-32768))
        for r in range(_ROWS))

    def body(_, carry):
        res, bitval = carry
        cands = [res[r] | bitval for r in range(_ROWS)]
        cnts = _count4(ref, cands)
        res = tuple(
            jnp.where(cnts[r] >= targets[r], cands[r], res[r])
            for r in range(_ROWS))
        return res, lax.shift_right_logical(bitval, 1)

    res, _ = lax.fori_loop(0, 15, body, (res0, jnp.int32(2**14)))
    return res


def _kwinners_body(x_ref, g_ref, o_ref, hi_ref, lo_ref):
    x = x_ref[...]
    pert = x / _TEMP + g_ref[...]
    s = lax.bitcast_convert_type(pert, jnp.int32)
    v = jnp.where(s < 0, s ^ jnp.int32(0x7FFFFFFF), s)
    hi_ref[...] = (v >> 16).astype(jnp.int16)
    lo_ref[...] = ((v & jnp.int32(0xFFFF)) ^ jnp.int32(0x8000)).astype(jnp.int16)

    hi_thr = _bisect4(hi_ref, [jnp.int32(_K)] * _ROWS)
    cgt = _count4(hi_ref, hi_thr, strict=True)
    k2 = [jnp.int32(_K) - cgt[r] for r in range(_ROWS)]

    for r in range(_ROWS):
        h16 = hi_thr[r].astype(jnp.int16)
        lo_ref[r] = jnp.where(hi_ref[r] == h16, lo_ref[r], jnp.int16(-32768))

    lo_thr = _bisect4(lo_ref, k2)

    for r in range(_ROWS):
        h16 = hi_thr[r].astype(jnp.int16)
        l16 = lo_thr[r].astype(jnp.int16)
        keep = (hi_ref[r] > h16) | ((hi_ref[r] == h16) & (lo_ref[r] >= l16))
        o_ref[r] = jnp.where(keep, x_ref[r], jnp.float32(0.0))


def kernel(x):
    g = _gumbel()
    x3 = x.reshape(_B, _SUB, _LANES)
    g3 = g.reshape(_B, _SUB, _LANES)
    out = pl.pallas_call(
        _kwinners_body,
        grid=(_B // _ROWS,),
        in_specs=[
            pl.BlockSpec((_ROWS, _SUB, _LANES), lambda i: (i, 0, 0)),
            pl.BlockSpec((_ROWS, _SUB, _LANES), lambda i: (i, 0, 0)),
        ],
        out_specs=pl.BlockSpec((_ROWS, _SUB, _LANES), lambda i: (i, 0, 0)),
        out_shape=jax.ShapeDtypeStruct((_B, _SUB, _LANES), jnp.float32),
        scratch_shapes=[
            pltpu.VMEM((_ROWS, _SUB, _LANES), jnp.int16),
            pltpu.VMEM((_ROWS, _SUB, _LANES), jnp.int16),
        ],
    )(x3, g3)
    return out.reshape(_B, _C, _H, _W)
